# baseline (speedup 1.0000x reference)
import functools

import jax
import jax.numpy as jnp
from jax import lax
from jax.experimental import pallas as pl
from jax.experimental.pallas import tpu as pltpu
from jax.experimental.pallas import tpu_sc as plsc

TOP_K = 8
NUM_EXPERTS = 64
HIDDEN_DIM = 4096
TOKEN_BLOCK = 1024


def _mm_block(hs_ref, wt_ref, out_ref):
    out_ref[...] = jnp.dot(
        hs_ref[...], wt_ref[...], preferred_element_type=jnp.float32
    )


def _tc_logits(hidden_states, wt):
    n_tokens = hidden_states.shape[0]
    blk = min(TOKEN_BLOCK, n_tokens)
    return pl.pallas_call(
        _mm_block,
        grid=(n_tokens // blk,),
        in_specs=[
            pl.BlockSpec((blk, HIDDEN_DIM), lambda i: (i, 0)),
            pl.BlockSpec((HIDDEN_DIM, NUM_EXPERTS), lambda i: (0, 0)),
        ],
        out_specs=pl.BlockSpec((blk, NUM_EXPERTS), lambda i: (i, 0)),
        out_shape=jax.ShapeDtypeStruct((n_tokens, NUM_EXPERTS), jnp.float32),
        compiler_params=pltpu.CompilerParams(
            dimension_semantics=("arbitrary",),
        ),
    )(hidden_states, wt)


def _take16(x, idx):
    dnums = lax.GatherDimensionNumbers(
        offset_dims=(), collapsed_slice_dims=(0,), start_index_map=(0,)
    )
    return lax.gather(
        x,
        idx[:, None],
        dnums,
        slice_sizes=(1,),
        mode=lax.GatherScatterMode.PROMISE_IN_BOUNDS,
    )


def _top8_row(lbuf, t, lane, shift, lo):
    ks, vs = [], []
    for j in range(4):
        kj = lbuf[pl.ds(t * NUM_EXPERTS + j * 16, 16)]
        vj = lane + (16 * j)
        sk, sv = plsc.sort_key_val(kj, vj, descending=True)
        ks.append(sk)
        vs.append(sv)

    def merge(ak, av, bk, bv):
        ck = jnp.where(lo, ak, _take16(bk, shift))
        cv = jnp.where(lo, av, _take16(bv, shift))
        return plsc.sort_key_val(ck, cv, descending=True)

    m0k, m0v = merge(ks[0], vs[0], ks[1], vs[1])
    m1k, m1v = merge(ks[2], vs[2], ks[3], vs[3])
    mk, mv = merge(m0k, m0v, m1k, m1v)

    e = jnp.exp(mk - jnp.max(mk))
    e = jnp.where(lo, e, 0.0)
    return e / jnp.sum(e), mv


def _sc_route(logits_flat, n_tokens):
    info = plsc.get_sparse_core_info()
    nw = info.num_cores * info.num_subcores
    per_w = n_tokens // nw
    mesh = plsc.VectorSubcoreMesh(core_axis_name="c", subcore_axis_name="s")

    @functools.partial(
        pl.kernel,
        mesh=mesh,
        out_type=[
            jax.ShapeDtypeStruct((n_tokens * TOP_K,), jnp.float32),
            jax.ShapeDtypeStruct((n_tokens * TOP_K,), jnp.int32),
        ],
        scratch_types=[
            pltpu.VMEM((per_w * NUM_EXPERTS,), jnp.float32),
            pltpu.VMEM((per_w * TOP_K,), jnp.float32),
            pltpu.VMEM((per_w * TOP_K,), jnp.int32),
        ],
        compiler_params=pltpu.CompilerParams(needs_layout_passes=False),
    )
    def route(logits_hbm, topv_hbm, topi_hbm, lbuf, vbuf, ibuf):
        wid = lax.axis_index("s") * info.num_cores + lax.axis_index("c")
        base = pl.multiple_of(wid * per_w, per_w)
        pltpu.sync_copy(
            logits_hbm.at[pl.ds(base * NUM_EXPERTS, per_w * NUM_EXPERTS)],
            lbuf,
        )
        lane = lax.iota(jnp.int32, 16)
        shift = (lane - 8) & 15
        lo = lane < 8

        def body(t, carry):
            va, ia = _top8_row(lbuf, 2 * t, lane, shift, lo)
            vb, ib = _top8_row(lbuf, 2 * t + 1, lane, shift, lo)
            vbuf[pl.ds(2 * t * TOP_K, 16)] = jnp.where(
                lo, va, _take16(vb, shift)
            )
            ibuf[pl.ds(2 * t * TOP_K, 16)] = jnp.where(
                lo, ia, _take16(ib, shift)
            )
            return carry

        lax.fori_loop(0, per_w // 2, body, 0)
        pltpu.sync_copy(vbuf, topv_hbm.at[pl.ds(base * TOP_K, per_w * TOP_K)])
        pltpu.sync_copy(ibuf, topi_hbm.at[pl.ds(base * TOP_K, per_w * TOP_K)])

    return route(logits_flat)


def kernel(hidden_states, weight):
    n_tokens = hidden_states.shape[0]
    logits = _tc_logits(hidden_states, weight.T)
    topv_flat, topi_flat = _sc_route(logits.reshape(-1), n_tokens)
    topv = topv_flat.reshape(n_tokens, TOP_K)
    topi = topi_flat.reshape(n_tokens, TOP_K)
    return (logits, topv, topi)


# speedup vs baseline: 1.5609x; 1.5609x over previous
import jax
import jax.numpy as jnp
from jax import lax
from jax.experimental import pallas as pl
from jax.experimental.pallas import tpu as pltpu

TOP_K = 8
NUM_EXPERTS = 64
HIDDEN_DIM = 4096
CHUNK = 512
N_BUF = 4


def _route_chunk(x, w, logits_out, topv_out, topi_out):
    lt = lax.dot_general(
        w, x, (((1,), (1,)), ((), ())), preferred_element_type=jnp.float32
    )
    r = lax.broadcasted_iota(jnp.int32, (NUM_EXPERTS, NUM_EXPERTS), 0)
    c = lax.broadcasted_iota(jnp.int32, (NUM_EXPERTS, NUM_EXPERTS), 1)
    eye = (r == c).astype(jnp.float32)
    logits_out[...] = lax.dot_general(
        lt, eye, (((0,), (0,)), ((), ())), preferred_element_type=jnp.float32
    )

    eiota = lax.broadcasted_iota(jnp.int32, (NUM_EXPERTS, CHUNK), 0)
    work = lt
    vals, idxs = [], []
    for _ in range(TOP_K):
        m = jnp.max(work, axis=0, keepdims=True)
        idx = jnp.min(
            jnp.where(work == m, eiota, NUM_EXPERTS), axis=0, keepdims=True
        )
        vals.append(m)
        idxs.append(idx)
        work = jnp.where(eiota == idx, -jnp.inf, work)
    topv = jnp.concatenate(vals, axis=0)
    topi = jnp.concatenate(idxs, axis=0)

    e = jnp.exp(topv - topv[0:1, :])
    topv_out[...] = e / jnp.sum(e, axis=0, keepdims=True)
    topi_out[...] = topi


def _router(
    hs_ref,
    w_ref,
    logits_ref,
    topv_ref,
    topi_ref,
    xbuf,
    lbuf,
    vbuf,
    ibuf,
    in_sems,
    l_sems,
    v_sems,
    i_sems,
):
    n_chunks = hs_ref.shape[0] // CHUNK

    def in_copy(chunk, slot):
        return pltpu.make_async_copy(
            hs_ref.at[pl.ds(chunk * CHUNK, CHUNK), :],
            xbuf.at[slot],
            in_sems.at[slot],
        )

    def out_copies(chunk, slot):
        return (
            pltpu.make_async_copy(
                lbuf.at[slot],
                logits_ref.at[pl.ds(chunk * CHUNK, CHUNK), :],
                l_sems.at[slot],
            ),
            pltpu.make_async_copy(
                vbuf.at[slot],
                topv_ref.at[:, pl.ds(chunk * CHUNK, CHUNK)],
                v_sems.at[slot],
            ),
            pltpu.make_async_copy(
                ibuf.at[slot],
                topi_ref.at[:, pl.ds(chunk * CHUNK, CHUNK)],
                i_sems.at[slot],
            ),
        )

    for b in range(N_BUF):
        in_copy(b, b).start()

    w = w_ref[...]

    def body(chunk, carry):
        slot = lax.rem(chunk, N_BUF)
        in_copy(chunk, slot).wait()

        @pl.when(chunk >= N_BUF)
        def _():
            for cp in out_copies(chunk - N_BUF, slot):
                cp.wait()

        _route_chunk(
            xbuf[slot], w, lbuf.at[slot], vbuf.at[slot], ibuf.at[slot]
        )
        for cp in out_copies(chunk, slot):
            cp.start()

        @pl.when(chunk + N_BUF < n_chunks)
        def _():
            in_copy(chunk + N_BUF, slot).start()

        return carry

    lax.fori_loop(0, n_chunks, body, 0)

    for b in range(N_BUF):
        chunk = n_chunks - N_BUF + b
        for cp in out_copies(chunk, lax.rem(chunk, N_BUF)):
            cp.wait()


def kernel(hidden_states, weight):
    n_tokens = hidden_states.shape[0]

    logits, topv_t, topi_t = pl.pallas_call(
        _router,
        in_specs=[
            pl.BlockSpec(memory_space=pl.ANY),
            pl.BlockSpec((NUM_EXPERTS, HIDDEN_DIM), lambda: (0, 0)),
        ],
        out_specs=[
            pl.BlockSpec(memory_space=pl.ANY),
            pl.BlockSpec(memory_space=pl.ANY),
            pl.BlockSpec(memory_space=pl.ANY),
        ],
        out_shape=[
            jax.ShapeDtypeStruct((n_tokens, NUM_EXPERTS), jnp.float32),
            jax.ShapeDtypeStruct((TOP_K, n_tokens), jnp.float32),
            jax.ShapeDtypeStruct((TOP_K, n_tokens), jnp.int32),
        ],
        scratch_shapes=[
            pltpu.VMEM((N_BUF, CHUNK, HIDDEN_DIM), jnp.float32),
            pltpu.VMEM((N_BUF, CHUNK, NUM_EXPERTS), jnp.float32),
            pltpu.VMEM((N_BUF, TOP_K, CHUNK), jnp.float32),
            pltpu.VMEM((N_BUF, TOP_K, CHUNK), jnp.int32),
            pltpu.SemaphoreType.DMA((N_BUF,)),
            pltpu.SemaphoreType.DMA((N_BUF,)),
            pltpu.SemaphoreType.DMA((N_BUF,)),
            pltpu.SemaphoreType.DMA((N_BUF,)),
        ],
    )(hidden_states, weight)
    return (logits, topv_t.T, topi_t.T)
